# baseline (device time: 114831 ns/iter reference)
import math

import jax
import jax.numpy as jnp
from jax import lax
from jax.experimental import pallas as pl
from jax.experimental.pallas import tpu as pltpu

N_DEV = 8
N_R = 4
N_L = 3
Q_CHUNK = 512


def kernel(q, k, v):
    S, D = q.shape
    scale = 1.0 / math.sqrt(D)
    n_chunks = S // Q_CHUNK

    def body(q_ref, k_ref, v_ref, out_ref, kvr_ref, kvl_ref, acc_ref, l_ref,
             send_r, recv_r, send_l, recv_l, credit_r, credit_l):
        my = lax.axis_index("i")
        left = lax.rem(my + N_DEV - 1, N_DEV)
        right = lax.rem(my + 1, N_DEV)

        barrier_sem = pltpu.get_barrier_semaphore()
        for nbr in (left, right):
            pl.semaphore_signal(barrier_sem, inc=1, device_id=(nbr,),
                                device_id_type=pl.DeviceIdType.MESH)
        pl.semaphore_wait(barrier_sem, 2)

        kb0 = k_ref[...].astype(jnp.bfloat16)
        vb0 = v_ref[...].astype(jnp.bfloat16)
        kvr_ref[0, 0] = kb0
        kvr_ref[0, 1] = vb0
        kvl_ref[0, 0] = kb0
        kvl_ref[0, 1] = vb0
        qv = q_ref[...].astype(jnp.bfloat16)
        l_ref[...] = jnp.zeros((S, 1), jnp.float32)
        acc_ref[...] = jnp.zeros((S, D), jnp.float32)

        def accumulate(kv_ref, cur):
            kb = kv_ref[cur, 0]
            vb = kv_ref[cur, 1]
            for c in range(n_chunks):
                rows = pl.ds(c * Q_CHUNK, Q_CHUNK)
                qc = qv[c * Q_CHUNK:(c + 1) * Q_CHUNK, :]
                s = lax.dot_general(
                    qc, kb, (((1,), (1,)), ((), ())),
                    preferred_element_type=jnp.float32,
                ) * scale
                p = jnp.exp(s.astype(jnp.bfloat16))
                l_ref[rows, :] = l_ref[rows, :] + jnp.sum(
                    p, axis=1, keepdims=True, dtype=jnp.float32)
                pv = lax.dot_general(
                    p, vb, (((1,), (0,)), ((), ())),
                    preferred_element_type=jnp.float32,
                )
                acc_ref[rows, :] = acc_ref[rows, :] + pv

        for t in range(N_R + 1):
            cur = t % 2
            nxt = (t + 1) % 2
            rdma_r = rdma_l = None
            if t < N_R:
                if t > 0:
                    pl.semaphore_wait(credit_r, 1)
                rdma_r = pltpu.make_async_remote_copy(
                    src_ref=kvr_ref.at[cur],
                    dst_ref=kvr_ref.at[nxt],
                    send_sem=send_r.at[cur],
                    recv_sem=recv_r.at[nxt],
                    device_id=(right,),
                    device_id_type=pl.DeviceIdType.MESH,
                )
                rdma_r.start()
            if t < N_L:
                if t > 0:
                    pl.semaphore_wait(credit_l, 1)
                rdma_l = pltpu.make_async_remote_copy(
                    src_ref=kvl_ref.at[cur],
                    dst_ref=kvl_ref.at[nxt],
                    send_sem=send_l.at[cur],
                    recv_sem=recv_l.at[nxt],
                    device_id=(left,),
                    device_id_type=pl.DeviceIdType.MESH,
                )
                rdma_l.start()

            accumulate(kvr_ref, cur)
            if 1 <= t <= N_L:
                accumulate(kvl_ref, cur)

            if t < N_R:
                rdma_r.wait_send()
                if t < N_R - 1:
                    pl.semaphore_signal(credit_r, inc=1, device_id=(left,),
                                        device_id_type=pl.DeviceIdType.MESH)
                rdma_r.wait_recv()
            if t < N_L:
                rdma_l.wait_send()
                if t < N_L - 1:
                    pl.semaphore_signal(credit_l, inc=1, device_id=(right,),
                                        device_id_type=pl.DeviceIdType.MESH)
                rdma_l.wait_recv()

        out_ref[...] = acc_ref[...] / l_ref[...]

    return pl.pallas_call(
        body,
        out_shape=jax.ShapeDtypeStruct((S, D), jnp.float32),
        in_specs=[pl.BlockSpec(memory_space=pltpu.VMEM)] * 3,
        out_specs=pl.BlockSpec(memory_space=pltpu.VMEM),
        scratch_shapes=[
            pltpu.VMEM((2, 2, S, D), jnp.bfloat16),
            pltpu.VMEM((2, 2, S, D), jnp.bfloat16),
            pltpu.VMEM((S, D), jnp.float32),
            pltpu.VMEM((S, 1), jnp.float32),
            pltpu.SemaphoreType.DMA((2,)),
            pltpu.SemaphoreType.DMA((2,)),
            pltpu.SemaphoreType.DMA((2,)),
            pltpu.SemaphoreType.DMA((2,)),
            pltpu.SemaphoreType.REGULAR,
            pltpu.SemaphoreType.REGULAR,
        ],
        compiler_params=pltpu.CompilerParams(collective_id=0),
    )(q, k, v)


# device time: 114814 ns/iter; 1.0001x vs baseline; 1.0001x over previous
import math

import jax
import jax.numpy as jnp
from jax import lax
from jax.experimental import pallas as pl
from jax.experimental.pallas import tpu as pltpu

N_DEV = 8
N_R = 4
N_L = 3
Q_CHUNK = 512


def kernel(q, k, v):
    S, D = q.shape
    scale = 1.0 / math.sqrt(D)
    n_chunks = S // Q_CHUNK

    def body(q_ref, k_ref, v_ref, out_ref, kvr_ref, kvl_ref, acc_ref, l_ref,
             send_r, recv_r, send_l, recv_l, credit_r, credit_l):
        my = lax.axis_index("i")
        left = lax.rem(my + N_DEV - 1, N_DEV)
        right = lax.rem(my + 1, N_DEV)

        barrier_sem = pltpu.get_barrier_semaphore()
        for nbr in (left, right):
            pl.semaphore_signal(barrier_sem, inc=1, device_id=(nbr,),
                                device_id_type=pl.DeviceIdType.MESH)
        pl.semaphore_wait(barrier_sem, 2)

        kb0 = k_ref[...].astype(jnp.bfloat16)
        vb0 = v_ref[...].astype(jnp.bfloat16)
        kvr_ref[0, 0] = kb0
        kvr_ref[0, 1] = vb0
        kvl_ref[0, 0] = kb0
        kvl_ref[0, 1] = vb0
        qv = q_ref[...].astype(jnp.bfloat16)
        l_ref[...] = jnp.zeros((S, 1), jnp.float32)
        acc_ref[...] = jnp.zeros((S, D), jnp.float32)

        def accumulate(kv_ref, cur):
            kb = kv_ref[cur, 0]
            vb = kv_ref[cur, 1]
            for c in range(n_chunks):
                rows = pl.ds(c * Q_CHUNK, Q_CHUNK)
                qc = qv[c * Q_CHUNK:(c + 1) * Q_CHUNK, :]
                s = lax.dot_general(
                    qc, kb, (((1,), (1,)), ((), ())),
                    preferred_element_type=jnp.float32,
                ) * scale
                p = jnp.exp(s)
                l_ref[rows, :] = l_ref[rows, :] + jnp.sum(
                    p, axis=1, keepdims=True)
                pv = lax.dot_general(
                    p.astype(jnp.bfloat16), vb, (((1,), (0,)), ((), ())),
                    preferred_element_type=jnp.float32,
                )
                acc_ref[rows, :] = acc_ref[rows, :] + pv

        for t in range(N_R + 1):
            cur = t % 2
            nxt = (t + 1) % 2
            rdma_r = rdma_l = None
            if t < N_R:
                if t > 0:
                    pl.semaphore_wait(credit_r, 1)
                rdma_r = pltpu.make_async_remote_copy(
                    src_ref=kvr_ref.at[cur],
                    dst_ref=kvr_ref.at[nxt],
                    send_sem=send_r.at[cur],
                    recv_sem=recv_r.at[nxt],
                    device_id=(right,),
                    device_id_type=pl.DeviceIdType.MESH,
                )
                rdma_r.start()
            if t < N_L:
                if t > 0:
                    pl.semaphore_wait(credit_l, 1)
                rdma_l = pltpu.make_async_remote_copy(
                    src_ref=kvl_ref.at[cur],
                    dst_ref=kvl_ref.at[nxt],
                    send_sem=send_l.at[cur],
                    recv_sem=recv_l.at[nxt],
                    device_id=(left,),
                    device_id_type=pl.DeviceIdType.MESH,
                )
                rdma_l.start()

            accumulate(kvr_ref, cur)
            if 1 <= t <= N_L:
                accumulate(kvl_ref, cur)

            if t < N_R:
                rdma_r.wait_send()
                if t < N_R - 1:
                    pl.semaphore_signal(credit_r, inc=1, device_id=(left,),
                                        device_id_type=pl.DeviceIdType.MESH)
                rdma_r.wait_recv()
            if t < N_L:
                rdma_l.wait_send()
                if t < N_L - 1:
                    pl.semaphore_signal(credit_l, inc=1, device_id=(right,),
                                        device_id_type=pl.DeviceIdType.MESH)
                rdma_l.wait_recv()

        out_ref[...] = acc_ref[...] / l_ref[...]

    return pl.pallas_call(
        body,
        out_shape=jax.ShapeDtypeStruct((S, D), jnp.float32),
        in_specs=[pl.BlockSpec(memory_space=pltpu.VMEM)] * 3,
        out_specs=pl.BlockSpec(memory_space=pltpu.VMEM),
        scratch_shapes=[
            pltpu.VMEM((2, 2, S, D), jnp.bfloat16),
            pltpu.VMEM((2, 2, S, D), jnp.bfloat16),
            pltpu.VMEM((S, D), jnp.float32),
            pltpu.VMEM((S, 1), jnp.float32),
            pltpu.SemaphoreType.DMA((2,)),
            pltpu.SemaphoreType.DMA((2,)),
            pltpu.SemaphoreType.DMA((2,)),
            pltpu.SemaphoreType.DMA((2,)),
            pltpu.SemaphoreType.REGULAR,
            pltpu.SemaphoreType.REGULAR,
        ],
        compiler_params=pltpu.CompilerParams(collective_id=0),
    )(q, k, v)
